# Initial kernel scaffold; baseline (speedup 1.0000x reference)
import jax
import jax.numpy as jnp
from jax.experimental import pallas as pl


def kernel(x, edge_index, batch, gin_w1_0, gin_b1_0, gin_g_0, gin_be_0, gin_w2_0, gin_b2_0, gin_w1_1, gin_b1_1, gin_g_1, gin_be_1, gin_w2_1, gin_b2_1, gin_w1_2, gin_b1_2, gin_g_2, gin_be_2, gin_w2_2, gin_b2_2, mlp_w1, mlp_b1, mlp_w2, mlp_b2):
    raise NotImplementedError("write your pallas kernel here")


# speedup vs baseline: 4.7066x; 4.7066x over previous
import functools

import jax
import jax.numpy as jnp
from jax import lax
from jax.experimental import pallas as pl
from jax.experimental.pallas import tpu as pltpu
from jax.experimental.pallas import tpu_sc as plsc

_NC = 2
_NS = 16
_BN_EPS = 1e-5


def _sc_agg(h, src, dst):
    N, D = h.shape
    (E,) = src.shape
    NW = _NC * _NS
    epw = E // NW
    CH = 80
    nch = epw // CH
    Np = ((N + 127) // 128) * 128
    rpt = Np // _NS
    ZR = 128
    mesh = plsc.VectorSubcoreMesh(core_axis_name="c", subcore_axis_name="s")

    @functools.partial(
        pl.kernel, mesh=mesh,
        out_type=jax.ShapeDtypeStruct((_NC * Np, D), jnp.float32),
        scratch_types=[
            pltpu.VMEM((CH,), jnp.int32),
            pltpu.VMEM((CH,), jnp.int32),
            pltpu.VMEM((CH, D), jnp.float32),
            pltpu.VMEM((ZR, D), jnp.float32),
            pltpu.VMEM_SHARED((Np, D), jnp.float32),
            pltpu.SemaphoreType.DMA,
        ],
    )
    def k(h_hbm, src_hbm, dst_hbm, out_hbm, sidx, didx, rows, zbuf, acc, sem):
        c = lax.axis_index("c")
        s = lax.axis_index("s")
        wid = s * _NC + c

        def zrow(i, carry):
            def zcol(j, carry2):
                zbuf[i, pl.ds(j * 16, 16)] = jnp.zeros((16,), jnp.float32)
                return carry2
            return lax.fori_loop(0, D // 16, zcol, carry)
        lax.fori_loop(0, ZR, zrow, 0)
        r0 = s * rpt
        for t in range(rpt // ZR):
            pltpu.sync_copy(zbuf, acc.at[pl.ds(r0 + t * ZR, ZR)])
        plsc.subcore_barrier()

        ebase = wid * epw

        def chunk(j, carry):
            off = ebase + j * CH
            pltpu.sync_copy(src_hbm.at[pl.ds(off, CH)], sidx)
            pltpu.sync_copy(dst_hbm.at[pl.ds(off, CH)], didx)
            pltpu.async_copy(h_hbm.at[sidx], rows, sem).wait()
            pltpu.sync_copy(rows, acc.at[didx], add=True)
            return carry
        lax.fori_loop(0, nch, chunk, 0)
        plsc.subcore_barrier()

        pltpu.sync_copy(acc.at[pl.ds(r0, rpt)],
                        out_hbm.at[pl.ds(c * Np + r0, rpt)])

    return k(h, src, dst).reshape(_NC, Np, D)


def _layer_call(h, agg2, w1f, b1f, w2, b2):
    N, D = h.shape
    bk = 2000
    nb = N // bk

    def kern(h_ref, a0_ref, a1_ref, w1_ref, b1_ref, w2_ref, b2_ref, o_ref):
        z = h_ref[...] + a0_ref[0] + a1_ref[0]
        t = jnp.dot(z, w1_ref[...], preferred_element_type=jnp.float32)
        t = jnp.maximum(t + b1_ref[...], 0.0)
        t = jnp.dot(t, w2_ref[...], preferred_element_type=jnp.float32)
        o_ref[...] = jnp.maximum(t + b2_ref[...], 0.0)

    return pl.pallas_call(
        kern,
        grid=(nb,),
        in_specs=[
            pl.BlockSpec((bk, D), lambda i: (i, 0)),
            pl.BlockSpec((1, bk, D), lambda i: (0, i, 0)),
            pl.BlockSpec((1, bk, D), lambda i: (1, i, 0)),
            pl.BlockSpec((D, D), lambda i: (0, 0)),
            pl.BlockSpec((1, D), lambda i: (0, 0)),
            pl.BlockSpec((D, D), lambda i: (0, 0)),
            pl.BlockSpec((1, D), lambda i: (0, 0)),
        ],
        out_specs=pl.BlockSpec((bk, D), lambda i: (i, 0)),
        out_shape=jax.ShapeDtypeStruct((N, D), jnp.float32),
    )(h, agg2, agg2, w1f, b1f, w2, b2)


def _final_call(h, agg2, w1f, b1f, w2, b2, batch3, G,
                mw1, mb1, mw2, mb2):
    N, D = h.shape
    D_OUT = mw2.shape[1]
    bk = 2000
    nb = N // bk

    def kern(h_ref, a0_ref, a1_ref, w1_ref, b1_ref, w2_ref, b2_ref, bt_ref,
             mw1_ref, mb1_ref, mw2_ref, mb2_ref, o_ref, pooled):
        i = pl.program_id(0)

        @pl.when(i == 0)
        def _():
            pooled[...] = jnp.zeros_like(pooled)

        z = h_ref[...] + a0_ref[0] + a1_ref[0]
        t = jnp.dot(z, w1_ref[...], preferred_element_type=jnp.float32)
        t = jnp.maximum(t + b1_ref[...], 0.0)
        t = jnp.dot(t, w2_ref[...], preferred_element_type=jnp.float32)
        h3 = jnp.maximum(t + b2_ref[...], 0.0)

        b = bt_ref[0, 0, :]
        onehot = (b[None, :] == lax.broadcasted_iota(jnp.int32, (G, bk), 0)
                  ).astype(jnp.float32)
        pooled[...] += jnp.dot(onehot, h3, preferred_element_type=jnp.float32)

        @pl.when(i == nb - 1)
        def _():
            y = jnp.dot(pooled[...], mw1_ref[...],
                        preferred_element_type=jnp.float32)
            y = jnp.maximum(y + mb1_ref[...], 0.0)
            o_ref[...] = jnp.dot(y, mw2_ref[...],
                                 preferred_element_type=jnp.float32) + mb2_ref[...]

    return pl.pallas_call(
        kern,
        grid=(nb,),
        in_specs=[
            pl.BlockSpec((bk, D), lambda i: (i, 0)),
            pl.BlockSpec((1, bk, D), lambda i: (0, i, 0)),
            pl.BlockSpec((1, bk, D), lambda i: (1, i, 0)),
            pl.BlockSpec((D, D), lambda i: (0, 0)),
            pl.BlockSpec((1, D), lambda i: (0, 0)),
            pl.BlockSpec((D, D), lambda i: (0, 0)),
            pl.BlockSpec((1, D), lambda i: (0, 0)),
            pl.BlockSpec((1, 1, bk), lambda i: (i, 0, 0)),
            pl.BlockSpec((D, D), lambda i: (0, 0)),
            pl.BlockSpec((1, D), lambda i: (0, 0)),
            pl.BlockSpec((D, D_OUT), lambda i: (0, 0)),
            pl.BlockSpec((1, D_OUT), lambda i: (0, 0)),
        ],
        out_specs=pl.BlockSpec((G, D_OUT), lambda i: (0, 0)),
        out_shape=jax.ShapeDtypeStruct((G, D_OUT), jnp.float32),
        scratch_shapes=[pltpu.VMEM((G, D), jnp.float32)],
    )(h, agg2, agg2, w1f, b1f, w2, b2, batch3,
      mw1, mb1, mw2, mb2)


def kernel(x, edge_index, batch,
           gin_w1_0, gin_b1_0, gin_g_0, gin_be_0, gin_w2_0, gin_b2_0,
           gin_w1_1, gin_b1_1, gin_g_1, gin_be_1, gin_w2_1, gin_b2_1,
           gin_w1_2, gin_b1_2, gin_g_2, gin_be_2, gin_w2_2, gin_b2_2,
           mlp_w1, mlp_b1, mlp_w2, mlp_b2):
    N, D = x.shape
    G = 64
    bk = 2000
    nb = N // bk
    src = edge_index[0]
    dst = edge_index[1]
    batch3 = batch.reshape(nb, 1, bk)

    params = []
    for (w1, b1, g, be, w2, b2) in (
        (gin_w1_0, gin_b1_0, gin_g_0, gin_be_0, gin_w2_0, gin_b2_0),
        (gin_w1_1, gin_b1_1, gin_g_1, gin_be_1, gin_w2_1, gin_b2_1),
        (gin_w1_2, gin_b1_2, gin_g_2, gin_be_2, gin_w2_2, gin_b2_2),
    ):
        scale = g / jnp.sqrt(1.0 + _BN_EPS)
        w1f = w1 * scale[None, :]
        b1f = (b1 * scale + be)[None, :]
        params.append((w1f, b1f, w2, b2[None, :]))

    h = x
    for i in range(2):
        agg2 = _sc_agg(h, src, dst)
        w1f, b1f, w2, b2 = params[i]
        h = _layer_call(h, agg2, w1f, b1f, w2, b2)

    agg2 = _sc_agg(h, src, dst)
    w1f, b1f, w2, b2 = params[2]
    return _final_call(h, agg2, w1f, b1f, w2, b2, batch3, G,
                       mlp_w1, mlp_b1[None, :], mlp_w2, mlp_b2[None, :])
